# baseline (speedup 1.0000x reference)
import numpy as np
import jax
import jax.numpy as jnp
from jax.experimental import pallas as pl
from jax.experimental.pallas import tpu as pltpu

_B = 1024
_BB = 128
_NEG = -1e30


def _moe_body(x_ref, M1_ref, b1_ref, M2_ref, b2_ref, Wg_ref, bg_ref,
              We1_ref, be1_ref, We2_ref, be2_ref, Ws_ref, bs_ref, out_ref):
    f32 = jnp.float32
    Bb = x_ref.shape[0]
    x = x_ref[...]

    X5 = jnp.stack(
        [jnp.concatenate([x[:, y + dy, :] for dy in range(5)], axis=-1)
         for y in range(24)], axis=1)
    Y1 = jax.lax.dot_general(
        X5.reshape(Bb * 24, 140), M1_ref[...],
        (((1,), (0,)), ((), ())), preferred_element_type=f32)
    Y1 = jnp.maximum(Y1 + b1_ref[...], 0.0)

    r1 = Y1.reshape(Bb, 24, 384)
    xp = jnp.maximum(r1[:, :, :192], r1[:, :, 192:])
    pooled = jnp.max(xp.reshape(Bb, 12, 2, 192), axis=2)

    X3 = jnp.stack(
        [jnp.concatenate([pooled[:, y + dy, :] for dy in range(3)], axis=-1)
         for y in range(10)], axis=1)
    Y2 = jax.lax.dot_general(
        X3.reshape(Bb * 10, 576), M2_ref[...],
        (((1,), (0,)), ((), ())), preferred_element_type=f32)
    Y2 = jnp.maximum(Y2 + b2_ref[...], 0.0)
    H3 = Y2.reshape(Bb, 10, 320)

    g = bg_ref[...]
    S1 = be1_ref[...]
    for y in range(10):
        hy = H3[:, y, :]
        g = g + jax.lax.dot_general(hy, Wg_ref[y], (((1,), (0,)), ((), ())),
                                    preferred_element_type=f32)
        S1 = S1 + jax.lax.dot_general(hy, We1_ref[y], (((1,), (0,)), ((), ())),
                                      preferred_element_type=f32)
    eh = jnp.tanh(S1)

    m = g
    vs, ohs = [], []
    for _ in range(3):
        v = jnp.max(m, axis=1, keepdims=True)
        eqf = jnp.where(m >= v, 1.0, 0.0)
        notbefore = jnp.ones((Bb, 1), dtype=f32)
        cols = []
        for e in range(5):
            cur = eqf[:, e:e + 1] * notbefore
            cols.append(cur)
            notbefore = notbefore * (1.0 - eqf[:, e:e + 1])
        oh = jnp.concatenate(cols, axis=1)
        vs.append(v)
        ohs.append(oh)
        m = m + oh * _NEG
    es = [jnp.exp(v - vs[0]) for v in vs]
    denom = es[0] + es[1] + es[2]
    gates = (ohs[0] * es[0] + ohs[1] * es[1] + ohs[2] * es[2]) / denom

    moe = jnp.zeros((Bb, 128), dtype=f32)
    for e in range(5):
        eo = jnp.tanh(
            jax.lax.dot_general(eh[:, e * 128:(e + 1) * 128], We2_ref[e],
                                (((1,), (0,)), ((), ())),
                                preferred_element_type=f32)
            + be2_ref[e:e + 1, :])
        moe = moe + gates[:, e:e + 1] * eo

    logits = jax.lax.dot_general(moe, Ws_ref[...], (((1,), (0,)), ((), ())),
                                 preferred_element_type=f32) + bs_ref[...]
    mx = jnp.max(logits, axis=1, keepdims=True)
    ex = jnp.exp(logits - mx)
    out_ref[...] = ex / jnp.sum(ex, axis=1, keepdims=True)


def _band_matrices(W1, W2):
    o, dy, dx, xo = np.meshgrid(np.arange(16), np.arange(5), np.arange(5),
                                np.arange(24), indexing='ij')
    rows = (dy * 28 + xo + dx).ravel()
    cols = ((xo % 2) * 192 + (xo // 2) * 16 + o).ravel()
    M1 = jnp.zeros((140, 384), jnp.float32).at[rows, cols].set(
        W1[:, 0][o.ravel(), dy.ravel(), dx.ravel()])

    o2, c2, dy2, dx2, xo2 = np.meshgrid(np.arange(32), np.arange(16),
                                        np.arange(3), np.arange(3),
                                        np.arange(10), indexing='ij')
    rows2 = (dy2 * 192 + (xo2 + dx2) * 16 + c2).ravel()
    cols2 = (xo2 * 32 + o2).ravel()
    M2 = jnp.zeros((576, 320), jnp.float32).at[rows2, cols2].set(
        W2[o2.ravel(), c2.ravel(), dy2.ravel(), dx2.ravel()])
    return M1, M2


def _flat_perm():
    y, x, o = np.meshgrid(np.arange(10), np.arange(10), np.arange(32),
                          indexing='ij')
    I = np.empty(3200, np.int32)
    I[(y * 320 + x * 32 + o).ravel()] = (o * 100 + y * 10 + x).ravel()
    return I


_I_FLAT = _flat_perm()


def kernel(x, W1, b1, W2, b2, Wg, bg, We1, be1, We2, be2, Ws, bs):
    f32 = jnp.float32
    xs = x.reshape(_B, 28, 28)
    M1, M2 = _band_matrices(W1, W2)
    b1rep = jnp.tile(b1, 24).reshape(1, 384)
    b2rep = jnp.tile(b2, 10).reshape(1, 320)
    Wg_p = Wg[_I_FLAT].reshape(10, 320, 5)
    We1_p = We1[:, _I_FLAT, :].transpose(1, 0, 2).reshape(10, 320, 640)
    be1f = be1.reshape(1, 640)
    bg2 = bg.reshape(1, 5)
    bs2 = bs.reshape(1, 10)

    grid = (_B // _BB,)
    out = pl.pallas_call(
        _moe_body,
        grid=grid,
        in_specs=[
            pl.BlockSpec((_BB, 28, 28), lambda i: (i, 0, 0)),
            pl.BlockSpec((140, 384), lambda i: (0, 0)),
            pl.BlockSpec((1, 384), lambda i: (0, 0)),
            pl.BlockSpec((576, 320), lambda i: (0, 0)),
            pl.BlockSpec((1, 320), lambda i: (0, 0)),
            pl.BlockSpec((10, 320, 5), lambda i: (0, 0, 0)),
            pl.BlockSpec((1, 5), lambda i: (0, 0)),
            pl.BlockSpec((10, 320, 640), lambda i: (0, 0, 0)),
            pl.BlockSpec((1, 640), lambda i: (0, 0)),
            pl.BlockSpec((5, 128, 128), lambda i: (0, 0, 0)),
            pl.BlockSpec((5, 128), lambda i: (0, 0)),
            pl.BlockSpec((128, 10), lambda i: (0, 0)),
            pl.BlockSpec((1, 10), lambda i: (0, 0)),
        ],
        out_specs=pl.BlockSpec((_BB, 10), lambda i: (i, 0)),
        out_shape=jax.ShapeDtypeStruct((_B, 10), f32),
        compiler_params=pltpu.CompilerParams(
            dimension_semantics=("arbitrary",)),
    )(xs, M1, b1rep, M2, b2rep, Wg_p, bg2, We1_p, be1f, We2, be2, Ws, bs2)
    return out
